# baseline (speedup 1.0000x reference)
import functools

import jax
import jax.numpy as jnp
from jax import lax
from jax.experimental import pallas as pl
from jax.experimental.pallas import tpu as pltpu
from jax.experimental.pallas import tpu_sc as plsc

NUM_USERS = 20000
NUM_ITEMS = 20000
HIDDEN = 16
HEADS = 4
OUT_D = HIDDEN * HEADS
SROW = 80
DROW = 16
RANGE = 20000
R_ACC = 20096
DUMP = 20000
CHUNK = 128
NW = 32


def _proj_body(x_ref, w_ref, b_ref, o_ref):
    o_ref[...] = x_ref[...] @ w_ref[...].T + b_ref[...]


def _proj(x, p):
    n, _ = x.shape
    blk = 2000 if n % 2000 == 0 else 1000
    return pl.pallas_call(
        _proj_body,
        grid=(n // blk,),
        in_specs=[
            pl.BlockSpec((blk, x.shape[1]), lambda i: (i, 0)),
            pl.BlockSpec(p["W"].shape, lambda i: (0, 0)),
            pl.BlockSpec((1, HIDDEN), lambda i: (0, 0)),
        ],
        out_specs=pl.BlockSpec((blk, HIDDEN), lambda i: (i, 0)),
        out_shape=jax.ShapeDtypeStruct((n, HIDDEN), jnp.float32),
    )(x, p["W"], p["b"].reshape(1, HIDDEN))


def _feat_body(x_ref, w_ref, aa_ref, h_ref, ab_ref):
    h = x_ref[...] @ w_ref[...].T
    h_ref[...] = h
    ab_ref[...] = h @ aa_ref[...]


def _feat(x, p, blk=2000):
    n, in_d = x.shape
    eye = jnp.eye(HEADS, dtype=jnp.float32)
    a_src = (p["a_src"][0].T[None, :, :] * eye[:, None, :]).reshape(OUT_D, HEADS)
    a_dst = (p["a_dst"][0].T[None, :, :] * eye[:, None, :]).reshape(OUT_D, HEADS)
    aa = jnp.concatenate([a_src, a_dst], axis=1)
    return pl.pallas_call(
        _feat_body,
        grid=(n // blk,),
        in_specs=[
            pl.BlockSpec((blk, in_d), lambda i: (i, 0)),
            pl.BlockSpec((OUT_D, in_d), lambda i: (0, 0)),
            pl.BlockSpec((OUT_D, 2 * HEADS), lambda i: (0, 0)),
        ],
        out_specs=[
            pl.BlockSpec((blk, OUT_D), lambda i: (i, 0)),
            pl.BlockSpec((blk, 2 * HEADS), lambda i: (i, 0)),
        ],
        out_shape=[
            jax.ShapeDtypeStruct((n, OUT_D), jnp.float32),
            jax.ShapeDtypeStruct((n, 2 * HEADS), jnp.float32),
        ],
    )(x, p["W"], aa)


def _epi_body(a0_ref, a1_ref, h_ref, ab_ref, b_ref, o_ref):
    acc = a0_ref[0] + a1_ref[0]
    num = acc[:, :OUT_D]
    den = acc[:, OUT_D:OUT_D + HEADS]
    ab = ab_ref[...]
    a = ab[:, :HEADS] + ab[:, HEADS:]
    ps = jnp.exp(jnp.where(a >= 0, a, 0.2 * a))
    hm = h_ref[...]
    psb = jnp.repeat(ps, HIDDEN, axis=1)
    num = num + hm * psb
    den = den + ps
    denb = jnp.repeat(den + 1e-16, HIDDEN, axis=1)
    o_ref[...] = num / denb + b_ref[...]


def _epilogue(acc, h_self, ab_self, bias, blk=2000):
    return pl.pallas_call(
        _epi_body,
        grid=(RANGE // blk,),
        in_specs=[
            pl.BlockSpec((1, blk, SROW), lambda i: (0, i, 0)),
            pl.BlockSpec((1, blk, SROW), lambda i: (1, i, 0)),
            pl.BlockSpec((blk, OUT_D), lambda i: (i, 0)),
            pl.BlockSpec((blk, 2 * HEADS), lambda i: (i, 0)),
            pl.BlockSpec((1, OUT_D), lambda i: (0, 0)),
        ],
        out_specs=pl.BlockSpec((blk, OUT_D), lambda i: (i, 0)),
        out_shape=jax.ShapeDtypeStruct((RANGE, OUT_D), jnp.float32),
    )(acc, acc, h_self, ab_self, bias.reshape(1, OUT_D))


def _epi4(a_ref, b_ref, o_ref):
    o_ref[...] = a_ref[...] + b_ref[...]


PREP = 512


@functools.lru_cache(maxsize=None)
def _make_sc_conv(e_pad, lo, n_nodes):
    per_tile = e_pad // NW
    n_chunks = per_tile // CHUNK
    rows_per_tile = R_ACC // 16
    mesh = plsc.VectorSubcoreMesh(core_axis_name="c", subcore_axis_name="s")

    @functools.partial(
        pl.kernel,
        out_type=jax.ShapeDtypeStruct((2, R_ACC, SROW), jnp.float32),
        mesh=mesh,
        compiler_params=pltpu.CompilerParams(use_tc_tiling_on_sc=False),
        scratch_types=[
            pltpu.VMEM((2, CHUNK), jnp.int32),
            pltpu.VMEM((2, CHUNK), jnp.int32),
            pltpu.VMEM((2, CHUNK), jnp.int32),
            pltpu.VMEM((2, CHUNK), jnp.int32),
            pltpu.VMEM((2, CHUNK, SROW), jnp.float32),
            pltpu.VMEM((2, CHUNK, DROW), jnp.float32),
            pltpu.VMEM_SHARED((R_ACC, SROW), jnp.float32),
            pltpu.SemaphoreType.DMA,
            pltpu.SemaphoreType.DMA,
            pltpu.SemaphoreType.DMA,
            pltpu.SemaphoreType.DMA,
            pltpu.SemaphoreType.DMA,
            pltpu.SemaphoreType.DMA,
        ],
    )
    def sc_conv(src_hbm, dst_hbm, s_hbm, d_hbm, zero_hbm, out_hbm,
                srcv, dstv, didx, sidx, sbuf, dbuf, accum,
                sg0, sg1, sd0, sd1, ss0, ss1):
        c = lax.axis_index("c")
        s = lax.axis_index("s")
        wid = s * 2 + c
        r0 = s * rows_per_tile
        pltpu.sync_copy(zero_hbm.at[pl.ds(r0, rows_per_tile)],
                        accum.at[pl.ds(r0, rows_per_tile)])
        plsc.subcore_barrier()
        base_w = wid * per_tile
        sg = (sg0, sg1)
        sd = (sd0, sd1)
        ss = (ss0, ss1)

        def prep_issue(g, b):
            base = base_w + g * CHUNK
            pltpu.sync_copy(src_hbm.at[pl.ds(base, CHUNK)], srcv.at[b])
            pltpu.sync_copy(dst_hbm.at[pl.ds(base, CHUNK)], dstv.at[b])
            for v in range(CHUNK // 16):
                d = dstv[b, pl.ds(16 * v, 16)]
                valid = (d >= lo) & (d < lo + RANGE)
                sidx[b, pl.ds(16 * v, 16)] = jnp.where(valid, d - lo, DUMP)
                didx[b, pl.ds(16 * v, 16)] = jnp.where(valid, d, 0)
            pltpu.async_copy(s_hbm.at[srcv.at[b]], sbuf.at[b], sg[b])
            pltpu.async_copy(d_hbm.at[didx.at[b]], dbuf.at[b], sd[b])

        def wait_gathers(b):
            pltpu.make_async_copy(s_hbm.at[srcv.at[b]], sbuf.at[b], sg[b]).wait()
            pltpu.make_async_copy(d_hbm.at[didx.at[b]], dbuf.at[b], sd[b]).wait()

        def compute(b):
            def edge_body(i, carry):
                a = sbuf[b, i, pl.ds(OUT_D, 16)] + dbuf[b, i, pl.ds(0, 16)]
                e = jnp.where(a >= 0, a, 0.2 * a)
                p = jnp.exp(e)
                sbuf[b, i, pl.ds(OUT_D, 16)] = p
                for k in range(HEADS):
                    pk = p[k]
                    sbuf[b, i, pl.ds(16 * k, 16)] = (
                        sbuf[b, i, pl.ds(16 * k, 16)] * pk)
                return carry

            lax.fori_loop(0, CHUNK, edge_body, 0)

        def issue_scatter(b):
            pltpu.async_copy(sbuf.at[b], accum.at[sidx.at[b]], ss[b], add=True)

        def wait_scatter(b):
            pltpu.make_async_copy(sbuf.at[b], accum.at[sidx.at[b]], ss[b]).wait()

        prep_issue(0, 0)
        prep_issue(1, 1)
        wait_gathers(0)
        compute(0)
        issue_scatter(0)

        def pair_body(t, carry):
            wait_scatter(0)
            prep_issue(2 * t + 2, 0)
            wait_gathers(1)
            compute(1)
            issue_scatter(1)
            wait_scatter(1)
            prep_issue(2 * t + 3, 1)
            wait_gathers(0)
            compute(0)
            issue_scatter(0)
            return carry

        lax.fori_loop(0, n_chunks // 2 - 1, pair_body, 0)
        wait_scatter(0)
        wait_gathers(1)
        compute(1)
        issue_scatter(1)
        wait_scatter(1)
        plsc.subcore_barrier()
        pltpu.sync_copy(accum.at[pl.ds(r0, rows_per_tile)],
                        out_hbm.at[c, pl.ds(r0, rows_per_tile)])

    return sc_conv


def _gat(h, ab, edges, bias, lo, zero_acc):
    n = h.shape[0]
    e = edges.shape[1]
    e_pad = ((e + NW * CHUNK * 2 - 1) // (NW * CHUNK * 2)) * (NW * CHUNK * 2)
    pad = e_pad - e
    src = jnp.concatenate([edges[0], jnp.zeros((pad,), jnp.int32)])
    dst = jnp.concatenate([edges[1], jnp.full((pad,), -1, jnp.int32)])
    s_tab = jnp.concatenate(
        [h, ab[:, :HEADS], jnp.zeros((n, SROW - OUT_D - HEADS), jnp.float32)], axis=1)
    d_tab = jnp.concatenate(
        [ab[:, HEADS:], jnp.zeros((n, DROW - HEADS), jnp.float32)], axis=1)
    acc = _make_sc_conv(e_pad, lo, n)(src, dst, s_tab, d_tab, zero_acc)
    sl = slice(lo, lo + RANGE)
    return _epilogue(acc, h[sl], ab[sl], bias)


def kernel(user_x, item_x, user_factor_0, item_factor_0,
           edge_u_q_u, edge_i_q_i, edge_u_i, edge_i_u, params):
    zero_acc = jnp.zeros((R_ACC, SROW), jnp.float32)
    all_emb = jnp.concatenate([
        _proj(user_x, params["user_proj"]),
        _proj(item_x, params["item_proj"]),
        _proj(user_factor_0, params["user_factor_proj"]),
        _proj(item_factor_0, params["item_factor_proj"]),
    ], axis=0)

    h1, ab1 = _feat(all_emb, params["gat_u_q_u"])
    h2, ab2 = _feat(all_emb, params["gat_i_q_i"])
    H_u = _gat(h1, ab1, edge_u_q_u, params["gat_u_q_u"]["b"], 0, zero_acc)
    H_i = _gat(h2, ab2, edge_i_q_i, params["gat_i_q_i"]["b"], NUM_USERS, zero_acc)
    combined = jnp.concatenate([H_u, H_i], axis=0)

    h3, ab3 = _feat(combined, params["gat_i_u"])
    h4, ab4 = _feat(combined, params["gat_u_i"])
    H_hat_u = _gat(h3, ab3, edge_u_i, params["gat_i_u"]["b"], 0, zero_acc)
    H_hat_i = _gat(h4, ab4, edge_i_u, params["gat_u_i"]["b"], NUM_USERS, zero_acc)
    return H_hat_u, H_hat_i
